# Initial kernel scaffold; baseline (speedup 1.0000x reference)
import jax
import jax.numpy as jnp
from jax.experimental import pallas as pl


def kernel(f_t, W_enc, b_enc, codebook, W1, b1, W2, b2, W_dec, b_dec):
    raise NotImplementedError("write your pallas kernel here")


# speedup vs baseline: 1.6947x; 1.6947x over previous
import functools

import jax
import jax.numpy as jnp
from jax import lax
from jax.experimental import pallas as pl
from jax.experimental.pallas import tpu as pltpu

_B = 4096
_D_IN = 1024
_M = 32
_D = 64
_K = 1024
_H = 64
_BETA = 0.25

_BB_ENC = 512
_FB = 1024
_BB_HEAD = 256


def _enc_kernel(f_ref, w_ref, b_ref, o_ref):
    o_ref[...] = (
        jnp.dot(f_ref[...], w_ref[...], preferred_element_type=jnp.float32)
        + b_ref[...]
    )


def _vq_kernel(flat_ref, cbt_ref, cb_ref, zq_ref, idx_ref, cnt_ref, loss_ref):
    g = pl.program_id(0)

    flat = flat_ref[...]
    cbt = cbt_ref[...]
    cb_sq = jnp.sum(cbt * cbt, axis=0, keepdims=True)
    row_sq = jnp.sum(flat * flat, axis=1, keepdims=True)
    scores = (row_sq + cb_sq) - 2.0 * jnp.dot(
        flat, cbt, preferred_element_type=jnp.float32
    )

    rowmin = jnp.min(scores, axis=1, keepdims=True)
    iota = lax.broadcasted_iota(jnp.int32, (_FB, _K), 1)
    idx = jnp.min(
        jnp.where(scores == rowmin, iota, _K), axis=1, keepdims=True
    )
    onehot = (iota == idx).astype(jnp.float32)

    zq = jnp.dot(onehot, cb_ref[...], preferred_element_type=jnp.float32)
    zq_ref[...] = zq
    idx_ref[0, 0, :] = idx[:, 0]

    @pl.when(g == 0)
    def _init():
        cnt_ref[...] = jnp.zeros_like(cnt_ref)
        loss_ref[...] = jnp.zeros_like(loss_ref)

    cnt_ref[0:1, :] += jnp.sum(onehot, axis=0, keepdims=True)
    diff = zq - flat
    loss_ref[...] += jnp.sum(
        jnp.sum(diff * diff, axis=1, keepdims=True), axis=0, keepdims=True
    )


def _head_kernel(zq_ref, w1_ref, b1_ref, w2_ref, b2_ref, wd_ref, bd_ref,
                 cnt_ref, loss_ref, logits_ref, fhat_ref, vql_ref, perp_ref):
    i = pl.program_id(0)
    zq = zq_ref[...]
    h = jax.nn.gelu(
        jnp.dot(zq, w1_ref[...], preferred_element_type=jnp.float32)
        + b1_ref[...]
    )
    logits_ref[...] = (
        jnp.dot(h, w2_ref[...], preferred_element_type=jnp.float32)
        + b2_ref[...]
    )
    fhat_ref[...] = (
        jnp.dot(zq, wd_ref[...], preferred_element_type=jnp.float32)
        + bd_ref[...]
    )

    @pl.when(i == 0)
    def _scalars():
        counts = cnt_ref[0:1, :]
        total = jnp.sum(counts, axis=1, keepdims=True)
        probs = counts / total
        ent = jnp.sum(probs * jnp.log(probs + 1e-10), axis=1, keepdims=True)
        perp_ref[...] = jnp.exp(-ent)
        vql_ref[...] = loss_ref[...] * ((1.0 + _BETA) / (_B * _M * _D))


@functools.partial(jax.jit, static_argnames=())
def kernel(f_t, W_enc, b_enc, codebook, W1, b1, W2, b2, W_dec, b_dec):
    z_e = pl.pallas_call(
        _enc_kernel,
        grid=(_B // _BB_ENC,),
        in_specs=[
            pl.BlockSpec((_BB_ENC, _D_IN), lambda i: (i, 0)),
            pl.BlockSpec((_D_IN, _M * _D), lambda i: (0, 0)),
            pl.BlockSpec((1, _M * _D), lambda i: (0, 0)),
        ],
        out_specs=pl.BlockSpec((_BB_ENC, _M * _D), lambda i: (i, 0)),
        out_shape=jax.ShapeDtypeStruct((_B, _M * _D), jnp.float32),
    )(f_t, W_enc, b_enc.reshape(1, _M * _D))

    flat = z_e.reshape(_B * _M, _D)
    n_vq = _B * _M // _FB

    zq_flat, idx_blocks, counts8, loss_sum = pl.pallas_call(
        _vq_kernel,
        grid=(n_vq,),
        in_specs=[
            pl.BlockSpec((_FB, _D), lambda g: (g, 0)),
            pl.BlockSpec((_D, _K), lambda g: (0, 0)),
            pl.BlockSpec((_K, _D), lambda g: (0, 0)),
        ],
        out_specs=[
            pl.BlockSpec((_FB, _D), lambda g: (g, 0)),
            pl.BlockSpec((1, 1, _FB), lambda g: (g, 0, 0)),
            pl.BlockSpec((8, _K), lambda g: (0, 0)),
            pl.BlockSpec((1, 1), lambda g: (0, 0)),
        ],
        out_shape=[
            jax.ShapeDtypeStruct((_B * _M, _D), jnp.float32),
            jax.ShapeDtypeStruct((n_vq, 1, _FB), jnp.int32),
            jax.ShapeDtypeStruct((8, _K), jnp.float32),
            jax.ShapeDtypeStruct((1, 1), jnp.float32),
        ],
    )(flat, codebook.T, codebook)

    z_q2 = zq_flat.reshape(_B, _M * _D)
    W2p = jnp.pad(W2, ((0, 0), (0, 128 - W2.shape[1])))
    b2p = jnp.pad(b2, (0, 128 - b2.shape[0])).reshape(1, 128)

    logits_p, f_hat, vql, perp = pl.pallas_call(
        _head_kernel,
        grid=(_B // _BB_HEAD,),
        in_specs=[
            pl.BlockSpec((_BB_HEAD, _M * _D), lambda i: (i, 0)),
            pl.BlockSpec((_M * _D, _H), lambda i: (0, 0)),
            pl.BlockSpec((1, _H), lambda i: (0, 0)),
            pl.BlockSpec((_H, 128), lambda i: (0, 0)),
            pl.BlockSpec((1, 128), lambda i: (0, 0)),
            pl.BlockSpec((_M * _D, _D_IN), lambda i: (0, 0)),
            pl.BlockSpec((1, _D_IN), lambda i: (0, 0)),
            pl.BlockSpec((8, _K), lambda i: (0, 0)),
            pl.BlockSpec((1, 1), lambda i: (0, 0)),
        ],
        out_specs=[
            pl.BlockSpec((_BB_HEAD, 128), lambda i: (i, 0)),
            pl.BlockSpec((_BB_HEAD, _D_IN), lambda i: (i, 0)),
            pl.BlockSpec((1, 1), lambda i: (0, 0)),
            pl.BlockSpec((1, 1), lambda i: (0, 0)),
        ],
        out_shape=[
            jax.ShapeDtypeStruct((_B, 128), jnp.float32),
            jax.ShapeDtypeStruct((_B, _D_IN), jnp.float32),
            jax.ShapeDtypeStruct((1, 1), jnp.float32),
            jax.ShapeDtypeStruct((1, 1), jnp.float32),
        ],
    )(z_q2, W1, b1.reshape(1, _H), W2p, b2p, W_dec, b_dec.reshape(1, _D_IN),
      counts8, loss_sum)

    logits = logits_p[:, : W2.shape[1]]
    z_q = zq_flat.reshape(_B, _M, _D)
    indices = idx_blocks.reshape(_B, _M)
    return (logits, f_hat, z_q, indices,
            vql.reshape(()), perp.reshape(()))
